# Initial kernel scaffold; baseline (speedup 1.0000x reference)
import jax
import jax.numpy as jnp
from jax.experimental import pallas as pl


def kernel(x, pos, edges, W1, b1, W2, b2):
    raise NotImplementedError("write your pallas kernel here")


# speedup vs baseline: 5.6081x; 5.6081x over previous
import functools

import jax
import jax.numpy as jnp
from jax import lax
from jax.experimental import pallas as pl
from jax.experimental.pallas import tpu as pltpu
from jax.experimental.pallas import tpu_sc as plsc

N = 10000
E = 160000
D = 256
H = 128
NP = 10240
EP = 163840
NC = 2
NS = 16
NW = NC * NS
EPT = EP // NW
CHUNK = 128
NCHUNK = EPT // CHUNK
RPT = NP // NS
BLK = 1024
BIG = 3.0e38

_sc_mesh = plsc.VectorSubcoreMesh(core_axis_name="c", subcore_axis_name="s")


def _stats_body(px_hbm, py_hbm, pz_hbm, row_hbm, col_hbm, stats_hbm,
                px, py, pz, rv, cv, tx0, tx1, tx2, tn0, tn1, tn2, cnt, bary):
    cid = lax.axis_index("c")
    sid = lax.axis_index("s")
    wid = cid * NS + sid
    base = wid * EPT

    pltpu.sync_copy(px_hbm, px)
    pltpu.sync_copy(py_hbm, py)
    pltpu.sync_copy(pz_hbm, pz)
    pltpu.sync_copy(row_hbm.at[pl.ds(base, EPT)], rv)
    pltpu.sync_copy(col_hbm.at[pl.ds(base, EPT)], cv)

    neg = jnp.full((16,), -BIG, jnp.float32)
    pos_ = jnp.full((16,), BIG, jnp.float32)
    zero = jnp.zeros((16,), jnp.float32)
    third = jnp.float32(1.0 / 3.0)

    @pl.loop(0, NP // 16)
    def _init(i):
        s = pl.ds(i * 16, 16)
        tx0[s] = neg
        tx1[s] = neg
        tx2[s] = neg
        tn0[s] = pos_
        tn1[s] = pos_
        tn2[s] = pos_
        cnt[s] = zero
        bary[s] = (px[s] + py[s] + pz[s]) * third

    lanes = lax.iota(jnp.int32, 16)

    @pl.loop(0, EPT // 16)
    def _edges(gi):
        s = pl.ds(gi * 16, 16)
        ro = rv[s]
        co = cv[s]
        vx = plsc.load_gather(px, [ro]) - plsc.load_gather(px, [co])
        vy = plsc.load_gather(py, [ro]) - plsc.load_gather(py, [co])
        vz = plsc.load_gather(pz, [ro]) - plsc.load_gather(pz, [co])
        one = jnp.ones((16,), jnp.float32)
        for l in range(16):
            m = lanes == l
            for ref, v in ((tx0, vx), (tx1, vy), (tx2, vz)):
                cur = plsc.load_gather(ref, [co], mask=m)
                plsc.store_scatter(ref, [co], jnp.maximum(cur, v), mask=m)
            for ref, v in ((tn0, vx), (tn1, vy), (tn2, vz)):
                cur = plsc.load_gather(ref, [co], mask=m)
                plsc.store_scatter(ref, [co], jnp.minimum(cur, v), mask=m)
            cur = plsc.load_gather(cnt, [co], mask=m)
            plsc.store_scatter(cnt, [co], cur + one, mask=m)

    for k, ref in enumerate((tx0, tx1, tx2, tn0, tn1, tn2, cnt, bary)):
        pltpu.sync_copy(ref, stats_hbm.at[wid, k])


_stats_call = functools.partial(
    pl.kernel,
    out_type=jax.ShapeDtypeStruct((NW, 8, NP), jnp.float32),
    mesh=_sc_mesh,
    compiler_params=pltpu.CompilerParams(needs_layout_passes=False),
    scratch_types=[
        pltpu.VMEM((NP,), jnp.float32),
        pltpu.VMEM((NP,), jnp.float32),
        pltpu.VMEM((NP,), jnp.float32),
        pltpu.VMEM((EPT,), jnp.int32),
        pltpu.VMEM((EPT,), jnp.int32),
        pltpu.VMEM((NP,), jnp.float32),
        pltpu.VMEM((NP,), jnp.float32),
        pltpu.VMEM((NP,), jnp.float32),
        pltpu.VMEM((NP,), jnp.float32),
        pltpu.VMEM((NP,), jnp.float32),
        pltpu.VMEM((NP,), jnp.float32),
        pltpu.VMEM((NP,), jnp.float32),
        pltpu.VMEM((NP,), jnp.float32),
    ],
)(_stats_body)


def _edge_body(g_hbm, row_hbm, col_hbm, b1_hbm, zero_hbm, out_hbm,
               acc, ridx, cidx, gr, gc, b1v, sem_r, sem_c):
    cid = lax.axis_index("c")
    sid = lax.axis_index("s")
    wid = cid * NS + sid
    base = wid * EPT

    pltpu.sync_copy(b1_hbm, b1v)
    pltpu.sync_copy(zero_hbm, acc.at[pl.ds(sid * RPT, RPT)])
    plsc.subcore_barrier()

    bvec = [b1v[pl.ds(k * 16, 16)] for k in range(H // 16)]

    @pl.loop(0, NCHUNK)
    def _chunk(j):
        off = base + j * CHUNK
        pltpu.sync_copy(row_hbm.at[pl.ds(off, CHUNK)], ridx)
        pltpu.sync_copy(col_hbm.at[pl.ds(off, CHUNK)], cidx)
        cp_r = pltpu.async_copy(g_hbm.at[ridx], gr, sem_r)
        cp_c = pltpu.async_copy(g_hbm.at[cidx], gc, sem_c)
        cp_r.wait()
        cp_c.wait()

        @pl.loop(0, CHUNK)
        def _erow(e):
            for k in range(H // 16):
                s = pl.ds(k * 16, 16)
                v = gr[e, s] - gc[e, s] + bvec[k]
                gr[e, s] = jnp.maximum(v, 0.0)

        pltpu.sync_copy(gr, acc.at[cidx], add=True)

    plsc.subcore_barrier()
    pltpu.sync_copy(acc.at[pl.ds(sid * RPT, RPT)],
                    out_hbm.at[cid, pl.ds(sid * RPT, RPT)])


_edge_call = functools.partial(
    pl.kernel,
    out_type=jax.ShapeDtypeStruct((NC, NP, H), jnp.float32),
    mesh=_sc_mesh,
    compiler_params=pltpu.CompilerParams(needs_layout_passes=False),
    scratch_types=[
        pltpu.VMEM_SHARED((NP, H), jnp.float32),
        pltpu.VMEM((CHUNK,), jnp.int32),
        pltpu.VMEM((CHUNK,), jnp.int32),
        pltpu.VMEM((CHUNK, H), jnp.float32),
        pltpu.VMEM((CHUNK, H), jnp.float32),
        pltpu.VMEM((H,), jnp.float32),
        pltpu.SemaphoreType.DMA,
        pltpu.SemaphoreType.DMA,
    ],
)(_edge_body)


def _prep_body(stats_ref, x_ref, w1x_ref, w1p_ref, g0_ref, g1_ref, cnt8_ref):
    s = stats_ref[...]
    rmax = jnp.max(s, axis=0)
    rmin = jnp.min(s, axis=0)
    rsum = jnp.sum(s, axis=0)
    cnt = rsum[:, 6:7]
    nonempty = cnt > 0.0
    tmax_m = jnp.where(nonempty, rmax[:, 0:3], 0.0)
    tmin_m = jnp.where(nonempty, rmin[:, 3:6], 0.0)
    bary = rmax[:, 7:8]
    n16 = jnp.concatenate(
        [tmin_m, tmax_m, bary, bary, bary,
         jnp.zeros((BLK, 7), jnp.float32)], axis=1)
    g = jnp.dot(x_ref[...], w1x_ref[...],
                preferred_element_type=jnp.float32,
                precision=lax.Precision.HIGHEST)
    g += jnp.dot(n16, w1p_ref[...],
                 preferred_element_type=jnp.float32,
                 precision=lax.Precision.HIGHEST)
    g0_ref[...] = g[:, :H]
    g1_ref[...] = g[:, H:]
    cnt8_ref[...] = rsum


def _prep_call(stats_t, x_pad, w1x, w1p):
    nblk = NP // BLK
    return pl.pallas_call(
        _prep_body,
        grid=(nblk,),
        in_specs=[
            pl.BlockSpec((NW, BLK, 8), lambda i: (0, i, 0)),
            pl.BlockSpec((BLK, D), lambda i: (i, 0)),
            pl.BlockSpec((D, D), lambda i: (0, 0)),
            pl.BlockSpec((16, D), lambda i: (0, 0)),
        ],
        out_specs=[
            pl.BlockSpec((BLK, H), lambda i: (i, 0)),
            pl.BlockSpec((BLK, H), lambda i: (i, 0)),
            pl.BlockSpec((BLK, 8), lambda i: (i, 0)),
        ],
        out_shape=[
            jax.ShapeDtypeStruct((NP, H), jnp.float32),
            jax.ShapeDtypeStruct((NP, H), jnp.float32),
            jax.ShapeDtypeStruct((NP, 8), jnp.float32),
        ],
    )(stats_t, x_pad, w1x, w1p)


def _final_body(p0_ref, p1_ref, cnt8_ref, w2_ref, b2_ref, out_ref):
    hs = jnp.concatenate(
        [jnp.sum(p0_ref[...], axis=0), jnp.sum(p1_ref[...], axis=0)], axis=1)
    o = jnp.dot(hs, w2_ref[...],
                preferred_element_type=jnp.float32,
                precision=lax.Precision.HIGHEST)
    out_ref[...] = o + cnt8_ref[:, 6:7] * b2_ref[...]


def _final_call(p0, p1, cnt8, w2, b2row):
    nblk = NP // BLK
    return pl.pallas_call(
        _final_body,
        grid=(nblk,),
        in_specs=[
            pl.BlockSpec((NC, BLK, H), lambda i: (0, i, 0)),
            pl.BlockSpec((NC, BLK, H), lambda i: (0, i, 0)),
            pl.BlockSpec((BLK, 8), lambda i: (i, 0)),
            pl.BlockSpec((D, D), lambda i: (0, 0)),
            pl.BlockSpec((1, D), lambda i: (0, 0)),
        ],
        out_specs=pl.BlockSpec((BLK, D), lambda i: (i, 0)),
        out_shape=jax.ShapeDtypeStruct((NP, D), jnp.float32),
    )(p0, p1, cnt8, w2, b2row)


def kernel(x, pos, edges, W1, b1, W2, b2):
    edges = edges.astype(jnp.int32)
    row = jnp.concatenate(
        [edges[0], jnp.zeros((EP - E,), jnp.int32)])
    col = jnp.concatenate(
        [edges[1], jnp.full((EP - E,), N, jnp.int32)])
    posp = jnp.pad(pos.astype(jnp.float32), ((0, NP - N), (0, 0)))
    px, py, pz = posp[:, 0], posp[:, 1], posp[:, 2]
    x_pad = jnp.pad(x, ((0, NP - N), (0, 0)))

    stats = _stats_call(px, py, pz, row, col)
    stats_t = jnp.transpose(stats, (0, 2, 1))

    w1p = jnp.concatenate([W1[:9], jnp.zeros((7, D), jnp.float32)])
    g0, g1, cnt8 = _prep_call(stats_t, x_pad, W1[9:], w1p)

    zeros_blk = jnp.zeros((RPT, H), jnp.float32)
    p0 = _edge_call(g0, row, col, b1[:H], zeros_blk)
    p1 = _edge_call(g1, row, col, b1[H:], zeros_blk)

    out = _final_call(p0, p1, cnt8, W2, b2.reshape(1, D))
    return out[:N]
